# Initial kernel scaffold; baseline (speedup 1.0000x reference)
import jax
import jax.numpy as jnp
from jax.experimental import pallas as pl


def kernel(inputs):
    raise NotImplementedError("write your pallas kernel here")


# speedup vs baseline: 6.3907x; 6.3907x over previous
import functools

import jax
import jax.numpy as jnp
from jax import lax
from jax.experimental import pallas as pl
from jax.experimental.pallas import tpu as pltpu
from jax.experimental.pallas import tpu_sc as plsc

N_ROWS = 128
N_COLS = 32768
N_KEEP = 1024
L = 16
NBINS = 2048
NV_COL = N_COLS // L
NV_BIN = NBINS // L
N_WORKERS = 32
ROWS_PER_W = N_ROWS // N_WORKERS
def _monotone_key(v):
    sign = jnp.uint32(0x80000000)
    u = lax.bitcast_convert_type(v, jnp.uint32)
    return jnp.where(u >= sign, ~u, u | sign)


def _topn_body(in_hbm, out_hbm, row_v, cand_v, hist_v, out_v):
    wid = lax.axis_index("c") * 16 + lax.axis_index("s")
    iota = lax.iota(jnp.int32, L)
    neg_inf = jnp.float32(-jnp.inf)

    def do_row(k, _):
        r = wid * ROWS_PER_W + k
        pltpu.sync_copy(in_hbm.at[r], row_v)

        def clear_step(i, c):
            hist_v[pl.ds(i * L, L)] = jnp.zeros((L,), jnp.int32)
            return c
        lax.fori_loop(0, NV_BIN, clear_step, 0)

        ones = jnp.ones((L,), jnp.int32)

        def hist_step(i, c):
            key = _monotone_key(row_v[pl.ds(i * L, L)])
            bins = (key >> jnp.uint32(21)).astype(jnp.int32)
            plsc.addupdate_scatter(hist_v, [bins], ones)
            return c
        lax.fori_loop(0, NV_COL, hist_step, 0)

        def scan_step(i, carry):
            acc, best = carry
            v = hist_v[pl.ds(i * L, L)]
            cs = plsc.cumsum(v)
            a_incl = N_COLS - (acc + cs - v)
            binvec = i * L + iota
            cand = jnp.where(a_incl >= N_KEEP, binvec, -1)
            return acc + jnp.sum(v), jnp.maximum(best, jnp.max(cand))
        _, bstar = lax.fori_loop(0, NV_BIN, scan_step,
                                 (jnp.int32(0), jnp.int32(-1)))
        thresh = bstar.astype(jnp.uint32) << jnp.uint32(21)

        def comp_step(i, off):
            v = row_v[pl.ds(i * L, L)]
            key = _monotone_key(v)
            msk = key >= thresh
            mi = msk.astype(jnp.int32)
            idx = jnp.where(msk, off + plsc.cumsum(mi) - 1, N_COLS + iota)
            plsc.store_scatter(cand_v, [idx], v)
            return off + jnp.sum(mi)
        m = lax.fori_loop(0, NV_COL, comp_step, jnp.int32(0))

        p = m - 1
        p = p | (p >> 1)
        p = p | (p >> 2)
        p = p | (p >> 4)
        p = p | (p >> 8)
        p = p | (p >> 16)
        n = jnp.maximum(p + 1, jnp.int32(N_KEEP))
        nv = n >> 4
        lgn = (jnp.int32(10)
               + (n > 1024).astype(jnp.int32)
               + (n > 2048).astype(jnp.int32)
               + (n > 4096).astype(jnp.int32)
               + (n > 8192).astype(jnp.int32)
               + (n > 16384).astype(jnp.int32))
        n_levels = lgn - 4
        mv = m >> 4

        def leaf_step(i, c):
            v = plsc.load_gather(cand_v, [i * L + iota])
            v = jnp.where(i * L + iota >= m, neg_inf, v)
            cand_v[pl.ds(i * L, L)] = jnp.sort(v)
            return c
        lax.fori_loop(mv, nv, leaf_step, 0)

        def leaf_sorted(i, c):
            v = plsc.load_gather(cand_v, [i * L + iota])
            cand_v[pl.ds(i * L, L)] = jnp.sort(v)
            return c
        lax.fori_loop(0, mv, leaf_sorted, 0)

        npair = nv >> 1

        def level_body(lr16, c0):
            r16m1 = (jnp.int32(1) << lr16) - 1

            def st_a(t, c):
                p = t >> lr16
                i = t & r16m1
                base = (p << (lr16 + 1)) + i
                s2 = base + ((r16m1 - i) << 1) + 1
                a = cand_v[pl.ds(base * L, L)]
                rb = lax.rev(cand_v[pl.ds(s2 * L, L)], (0,))
                cand_v[pl.ds(base * L, L)] = jnp.minimum(a, rb)
                cand_v[pl.ds(s2 * L, L)] = lax.rev(jnp.maximum(a, rb), (0,))
                return c
            lax.fori_loop(0, npair, st_a, 0)

            def st_b(j, c):
                ld16 = lr16 - 1 - j
                d16 = jnp.int32(1) << ld16
                d16m1 = d16 - 1

                def st_b_step(t, c2):
                    s = ((t >> ld16) << (ld16 + 1)) + (t & d16m1)
                    s2 = s + d16
                    a = cand_v[pl.ds(s * L, L)]
                    b = cand_v[pl.ds(s2 * L, L)]
                    cand_v[pl.ds(s * L, L)] = jnp.minimum(a, b)
                    cand_v[pl.ds(s2 * L, L)] = jnp.maximum(a, b)
                    return c2
                lax.fori_loop(0, npair, st_b_step, 0)
                return c
            lax.fori_loop(0, lr16, st_b, 0)

            def st_c(i, c):
                sl = pl.ds(i * L, L)
                cand_v[sl] = jnp.sort(cand_v[sl])
                return c
            lax.fori_loop(0, nv, st_c, 0)
            return c0
        lax.fori_loop(0, n_levels, level_body, 0)

        def emit_step(j, c):
            out_v[pl.ds(j * L, L)] = lax.rev(
                cand_v[pl.ds((nv - 1 - j) * L, L)], (0,))
            return c
        lax.fori_loop(0, N_KEEP // L, emit_step, 0)
        pltpu.sync_copy(out_v, out_hbm.at[r])
        return 0

    lax.fori_loop(0, ROWS_PER_W, do_row, 0)


@jax.jit
def kernel(inputs):
    mesh = plsc.VectorSubcoreMesh(core_axis_name="c", subcore_axis_name="s")
    f = pl.kernel(
        _topn_body,
        out_type=jax.ShapeDtypeStruct((N_ROWS, N_KEEP), jnp.float32),
        mesh=mesh,
        compiler_params=pltpu.CompilerParams(needs_layout_passes=False),
        scratch_types=[
            pltpu.VMEM((N_COLS,), jnp.float32),
            pltpu.VMEM((N_COLS + L,), jnp.float32),
            pltpu.VMEM((NBINS,), jnp.int32),
            pltpu.VMEM((N_KEEP,), jnp.float32),
        ],
    )
    return f(inputs)


# speedup vs baseline: 16.8067x; 2.6299x over previous
import functools

import jax
import jax.numpy as jnp
from jax import lax
from jax.experimental import pallas as pl
from jax.experimental.pallas import tpu as pltpu
from jax.experimental.pallas import tpu_sc as plsc

N_ROWS = 128
N_COLS = 32768
N_KEEP = 1024
L = 16
NBINS = 2048
NV_COL = N_COLS // L
NV_BIN = NBINS // L
N_WORKERS = 32
ROWS_PER_W = N_ROWS // N_WORKERS
def _monotone_key(v):
    sign = jnp.uint32(0x80000000)
    u = lax.bitcast_convert_type(v, jnp.uint32)
    return jnp.where(u >= sign, ~u, u | sign)


def _topn_body(in_hbm, out_hbm, row_v, cand_v, hist_v, out_v):
    wid = lax.axis_index("c") * 16 + lax.axis_index("s")
    iota = lax.iota(jnp.int32, L)
    neg_inf = jnp.float32(-jnp.inf)

    def do_row(k, _):
        r = wid * ROWS_PER_W + k
        pltpu.sync_copy(in_hbm.at[r], row_v)

        @plsc.parallel_loop(0, NV_BIN, unroll=4)
        def _clear(i):
            hist_v[pl.ds(i * L, L)] = jnp.zeros((L,), jnp.int32)

        ones = jnp.ones((L,), jnp.int32)

        @plsc.parallel_loop(0, NV_COL, unroll=4)
        def _hist(i):
            key = _monotone_key(row_v[pl.ds(i * L, L)])
            bins = (key >> jnp.uint32(21)).astype(jnp.int32)
            plsc.addupdate_scatter(hist_v, [bins], ones)

        def scan_step(i, carry):
            acc, best = carry
            v = hist_v[pl.ds(i * L, L)]
            cs = plsc.cumsum(v)
            a_incl = N_COLS - (acc + cs - v)
            binvec = i * L + iota
            cand = jnp.where(a_incl >= N_KEEP, binvec, -1)
            return acc + jnp.sum(v), jnp.maximum(best, jnp.max(cand))
        _, bstar = lax.fori_loop(0, NV_BIN, scan_step,
                                 (jnp.int32(0), jnp.int32(-1)))
        thresh = bstar.astype(jnp.uint32) << jnp.uint32(21)

        @plsc.parallel_loop(0, NV_COL, unroll=2, carry=jnp.int32(0))
        def comp_loop(i, off):
            v = row_v[pl.ds(i * L, L)]
            key = _monotone_key(v)
            msk = key >= thresh
            mi = msk.astype(jnp.int32)
            cs = plsc.cumsum(mi)
            idx = jnp.where(msk, off + cs - 1, N_COLS + iota)
            plsc.store_scatter(cand_v, [idx], v)
            return off + jnp.max(cs)
        m = comp_loop

        p = m - 1
        p = p | (p >> 1)
        p = p | (p >> 2)
        p = p | (p >> 4)
        p = p | (p >> 8)
        p = p | (p >> 16)
        n = jnp.maximum(p + 1, jnp.int32(N_KEEP))
        nv = n >> 4
        lgn = (jnp.int32(10)
               + (n > 1024).astype(jnp.int32)
               + (n > 2048).astype(jnp.int32)
               + (n > 4096).astype(jnp.int32)
               + (n > 8192).astype(jnp.int32)
               + (n > 16384).astype(jnp.int32))
        n_levels = lgn - 4
        mv = m >> 4

        @plsc.parallel_loop(mv, nv)
        def _leaf_pad(i):
            v = plsc.load_gather(cand_v, [i * L + iota])
            v = jnp.where(i * L + iota >= m, neg_inf, v)
            cand_v[pl.ds(i * L, L)] = jnp.sort(v)

        @plsc.parallel_loop(0, mv)
        def _leaf(i):
            v = plsc.load_gather(cand_v, [i * L + iota])
            cand_v[pl.ds(i * L, L)] = jnp.sort(v)

        npair = nv >> 1

        def level_body(lr16, c0):
            r16m1 = (jnp.int32(1) << lr16) - 1

            @plsc.parallel_loop(0, npair)
            def _st_a(t):
                p = t >> lr16
                i = t & r16m1
                base = (p << (lr16 + 1)) + i
                s2 = base + ((r16m1 - i) << 1) + 1
                a = cand_v[pl.ds(base * L, L)]
                rb = lax.rev(cand_v[pl.ds(s2 * L, L)], (0,))
                cand_v[pl.ds(base * L, L)] = jnp.minimum(a, rb)
                cand_v[pl.ds(s2 * L, L)] = lax.rev(jnp.maximum(a, rb), (0,))

            def st_b(j, c):
                ld16 = lr16 - 1 - j
                d16 = jnp.int32(1) << ld16
                d16m1 = d16 - 1

                @plsc.parallel_loop(0, npair)
                def _st_b(t):
                    s = ((t >> ld16) << (ld16 + 1)) + (t & d16m1)
                    s2 = s + d16
                    a = cand_v[pl.ds(s * L, L)]
                    b = cand_v[pl.ds(s2 * L, L)]
                    cand_v[pl.ds(s * L, L)] = jnp.minimum(a, b)
                    cand_v[pl.ds(s2 * L, L)] = jnp.maximum(a, b)
                return c
            lax.fori_loop(0, lr16, st_b, 0)

            @plsc.parallel_loop(0, nv)
            def _st_c(i):
                sl = pl.ds(i * L, L)
                cand_v[sl] = jnp.sort(cand_v[sl])
            return c0
        lax.fori_loop(0, n_levels, level_body, 0)

        @plsc.parallel_loop(0, N_KEEP // L, unroll=4)
        def _emit(j):
            out_v[pl.ds(j * L, L)] = lax.rev(
                cand_v[pl.ds((nv - 1 - j) * L, L)], (0,))
        pltpu.sync_copy(out_v, out_hbm.at[r])
        return 0

    lax.fori_loop(0, ROWS_PER_W, do_row, 0)


@jax.jit
def kernel(inputs):
    mesh = plsc.VectorSubcoreMesh(core_axis_name="c", subcore_axis_name="s")
    f = pl.kernel(
        _topn_body,
        out_type=jax.ShapeDtypeStruct((N_ROWS, N_KEEP), jnp.float32),
        mesh=mesh,
        compiler_params=pltpu.CompilerParams(needs_layout_passes=False),
        scratch_types=[
            pltpu.VMEM((N_COLS,), jnp.float32),
            pltpu.VMEM((N_COLS + L,), jnp.float32),
            pltpu.VMEM((NBINS,), jnp.int32),
            pltpu.VMEM((N_KEEP,), jnp.float32),
        ],
    )
    return f(inputs)


# speedup vs baseline: 20.4108x; 1.2144x over previous
import functools

import jax
import jax.numpy as jnp
from jax import lax
from jax.experimental import pallas as pl
from jax.experimental.pallas import tpu as pltpu
from jax.experimental.pallas import tpu_sc as plsc

N_ROWS = 128
N_COLS = 32768
N_KEEP = 1024
L = 16
NBINS = 2048
NV_COL = N_COLS // L
NV_BIN = NBINS // L
N_WORKERS = 32
ROWS_PER_W = N_ROWS // N_WORKERS
def _monotone_key(v):
    sign = jnp.uint32(0x80000000)
    u = lax.bitcast_convert_type(v, jnp.uint32)
    return jnp.where(u >= sign, ~u, u | sign)


def _topn_body(in_hbm, out_hbm, row_v, cand_v, hist_v, out_v):
    wid = lax.axis_index("c") * 16 + lax.axis_index("s")
    iota = lax.iota(jnp.int32, L)
    neg_inf = jnp.float32(-jnp.inf)

    def do_row(k, _):
        r = wid * ROWS_PER_W + k
        pltpu.sync_copy(in_hbm.at[r], row_v)

        @plsc.parallel_loop(0, NV_BIN, unroll=4)
        def _clear(i):
            hist_v[pl.ds(i * L, L)] = jnp.zeros((L,), jnp.int32)

        ones = jnp.ones((L,), jnp.int32)

        @plsc.parallel_loop(0, NV_COL, unroll=8)
        def _hist(i):
            key = _monotone_key(row_v[pl.ds(i * L, L)])
            bins = (key >> jnp.uint32(21)).astype(jnp.int32)
            plsc.addupdate_scatter(hist_v, [bins], ones)

        def scan_step(i, carry):
            acc, best = carry
            v = hist_v[pl.ds(i * L, L)]
            cs = plsc.cumsum(v)
            a_incl = N_COLS - (acc + cs - v)
            binvec = i * L + iota
            cand = jnp.where(a_incl >= N_KEEP, binvec, -1)
            return acc + jnp.sum(v), jnp.maximum(best, jnp.max(cand))
        _, bstar = lax.fori_loop(0, NV_BIN, scan_step,
                                 (jnp.int32(0), jnp.int32(-1)))
        thresh = bstar.astype(jnp.uint32) << jnp.uint32(21)

        @plsc.parallel_loop(0, NV_COL, unroll=4, carry=jnp.int32(0))
        def comp_loop(i, off):
            v = row_v[pl.ds(i * L, L)]
            key = _monotone_key(v)
            msk = key >= thresh
            mi = msk.astype(jnp.int32)
            cs = plsc.cumsum(mi)
            idx = jnp.where(msk, off + cs - 1, N_COLS + iota)
            plsc.store_scatter(cand_v, [idx], v)
            return off + jnp.max(cs)
        m = comp_loop

        p = m - 1
        p = p | (p >> 1)
        p = p | (p >> 2)
        p = p | (p >> 4)
        p = p | (p >> 8)
        p = p | (p >> 16)
        n = jnp.maximum(p + 1, jnp.int32(N_KEEP))
        nv = n >> 4
        lgn = (jnp.int32(10)
               + (n > 1024).astype(jnp.int32)
               + (n > 2048).astype(jnp.int32)
               + (n > 4096).astype(jnp.int32)
               + (n > 8192).astype(jnp.int32)
               + (n > 16384).astype(jnp.int32))
        n_levels = lgn - 4
        mv = m >> 4

        @plsc.parallel_loop(mv, nv, unroll=2)
        def _leaf_pad(i):
            v = plsc.load_gather(cand_v, [i * L + iota])
            v = jnp.where(i * L + iota >= m, neg_inf, v)
            cand_v[pl.ds(i * L, L)] = jnp.sort(v)

        @plsc.parallel_loop(0, mv, unroll=2)
        def _leaf(i):
            v = plsc.load_gather(cand_v, [i * L + iota])
            cand_v[pl.ds(i * L, L)] = jnp.sort(v)

        npair = nv >> 1

        def level_body(lr16, c0):
            r16m1 = (jnp.int32(1) << lr16) - 1

            @plsc.parallel_loop(0, npair, unroll=2)
            def _st_a(t):
                p = t >> lr16
                i = t & r16m1
                base = (p << (lr16 + 1)) + i
                s2 = base + ((r16m1 - i) << 1) + 1
                a = cand_v[pl.ds(base * L, L)]
                rb = lax.rev(cand_v[pl.ds(s2 * L, L)], (0,))
                cand_v[pl.ds(base * L, L)] = jnp.minimum(a, rb)
                cand_v[pl.ds(s2 * L, L)] = lax.rev(jnp.maximum(a, rb), (0,))

            def st_b(j, c):
                ld16 = lr16 - 1 - j
                d16 = jnp.int32(1) << ld16
                d16m1 = d16 - 1

                @plsc.parallel_loop(0, npair, unroll=2)
                def _st_b(t):
                    s = ((t >> ld16) << (ld16 + 1)) + (t & d16m1)
                    s2 = s + d16
                    a = cand_v[pl.ds(s * L, L)]
                    b = cand_v[pl.ds(s2 * L, L)]
                    cand_v[pl.ds(s * L, L)] = jnp.minimum(a, b)
                    cand_v[pl.ds(s2 * L, L)] = jnp.maximum(a, b)
                return c
            lax.fori_loop(0, lr16, st_b, 0)

            @plsc.parallel_loop(0, nv, unroll=2)
            def _st_c(i):
                sl = pl.ds(i * L, L)
                cand_v[sl] = jnp.sort(cand_v[sl])
            return c0
        lax.fori_loop(0, n_levels, level_body, 0)

        @plsc.parallel_loop(0, N_KEEP // L, unroll=4)
        def _emit(j):
            out_v[pl.ds(j * L, L)] = lax.rev(
                cand_v[pl.ds((nv - 1 - j) * L, L)], (0,))
        pltpu.sync_copy(out_v, out_hbm.at[r])
        return 0

    lax.fori_loop(0, ROWS_PER_W, do_row, 0)


@jax.jit
def kernel(inputs):
    mesh = plsc.VectorSubcoreMesh(core_axis_name="c", subcore_axis_name="s")
    f = pl.kernel(
        _topn_body,
        out_type=jax.ShapeDtypeStruct((N_ROWS, N_KEEP), jnp.float32),
        mesh=mesh,
        compiler_params=pltpu.CompilerParams(needs_layout_passes=False),
        scratch_types=[
            pltpu.VMEM((N_COLS,), jnp.float32),
            pltpu.VMEM((N_COLS + L,), jnp.float32),
            pltpu.VMEM((NBINS,), jnp.int32),
            pltpu.VMEM((N_KEEP,), jnp.float32),
        ],
    )
    return f(inputs)


# speedup vs baseline: 21.9753x; 1.0766x over previous
import functools

import jax
import jax.numpy as jnp
from jax import lax
from jax.experimental import pallas as pl
from jax.experimental.pallas import tpu as pltpu
from jax.experimental.pallas import tpu_sc as plsc

N_ROWS = 128
N_COLS = 32768
N_KEEP = 1024
L = 16
NBINS = 2048
NV_COL = N_COLS // L
NV_BIN = NBINS // L
N_WORKERS = 32
ROWS_PER_W = N_ROWS // N_WORKERS
def _monotone_key(v):
    sign = jnp.uint32(0x80000000)
    u = lax.bitcast_convert_type(v, jnp.uint32)
    return jnp.where(u >= sign, ~u, u | sign)


def _topn_body(in_hbm, out_hbm, row_v, cand_v, hist_v, out_v):
    wid = lax.axis_index("c") * 16 + lax.axis_index("s")
    iota = lax.iota(jnp.int32, L)
    neg_inf = jnp.float32(-jnp.inf)

    def do_row(k, _):
        r = wid * ROWS_PER_W + k
        pltpu.sync_copy(in_hbm.at[r], row_v)

        @plsc.parallel_loop(0, NV_BIN, unroll=4)
        def _clear(i):
            hist_v[pl.ds(i * L, L)] = jnp.zeros((L,), jnp.int32)

        ones = jnp.ones((L,), jnp.int32)

        @plsc.parallel_loop(0, NV_COL, unroll=8)
        def _hist(i):
            key = _monotone_key(row_v[pl.ds(i * L, L)])
            bins = (key >> jnp.uint32(21)).astype(jnp.int32)
            plsc.addupdate_scatter(hist_v, [bins], ones)

        @plsc.parallel_loop(0, NV_BIN, unroll=2,
                            carry=(jnp.int32(0), jnp.int32(-1)))
        def scan_loop(i, carry):
            acc, best = carry
            v = hist_v[pl.ds(i * L, L)]
            cs = plsc.cumsum(v)
            a_incl = N_COLS - (acc + cs - v)
            binvec = i * L + iota
            cand = jnp.where(a_incl >= N_KEEP, binvec, -1)
            return acc + jnp.max(cs), jnp.maximum(best, jnp.max(cand))
        _, bstar = scan_loop
        thresh = bstar.astype(jnp.uint32) << jnp.uint32(21)

        @plsc.parallel_loop(0, NV_COL, unroll=4, carry=jnp.int32(0))
        def comp_loop(i, off):
            v = row_v[pl.ds(i * L, L)]
            key = _monotone_key(v)
            msk = key >= thresh
            mi = msk.astype(jnp.int32)
            cs = plsc.cumsum(mi)
            idx = jnp.where(msk, off + cs - 1, N_COLS + iota)
            plsc.store_scatter(cand_v, [idx], v)
            return off + jnp.max(cs)
        m = comp_loop

        p = m - 1
        p = p | (p >> 1)
        p = p | (p >> 2)
        p = p | (p >> 4)
        p = p | (p >> 8)
        p = p | (p >> 16)
        n = jnp.maximum(p + 1, jnp.int32(N_KEEP))
        nv = n >> 4
        lgn = (jnp.int32(10)
               + (n > 1024).astype(jnp.int32)
               + (n > 2048).astype(jnp.int32)
               + (n > 4096).astype(jnp.int32)
               + (n > 8192).astype(jnp.int32)
               + (n > 16384).astype(jnp.int32))
        n_levels = lgn - 4
        mv = m >> 4

        @plsc.parallel_loop(mv, nv, unroll=4)
        def _leaf_pad(i):
            v = jnp.where(i * L + iota >= m, neg_inf, cand_v[pl.ds(i * L, L)])
            cand_v[pl.ds(i * L, L)] = jnp.sort(v)

        @plsc.parallel_loop(0, mv, unroll=4)
        def _leaf(i):
            cand_v[pl.ds(i * L, L)] = jnp.sort(cand_v[pl.ds(i * L, L)])

        npair = nv >> 1

        def level_body(lr16, c0):
            r16m1 = (jnp.int32(1) << lr16) - 1

            @plsc.parallel_loop(0, npair, unroll=4)
            def _st_a(t):
                p = t >> lr16
                i = t & r16m1
                base = (p << (lr16 + 1)) + i
                s2 = base + ((r16m1 - i) << 1) + 1
                a = cand_v[pl.ds(base * L, L)]
                rb = lax.rev(cand_v[pl.ds(s2 * L, L)], (0,))
                cand_v[pl.ds(base * L, L)] = jnp.minimum(a, rb)
                cand_v[pl.ds(s2 * L, L)] = lax.rev(jnp.maximum(a, rb), (0,))

            def st_b(j, c):
                ld16 = lr16 - 1 - j
                d16 = jnp.int32(1) << ld16
                d16m1 = d16 - 1

                @plsc.parallel_loop(0, npair, unroll=4)
                def _st_b(t):
                    s = ((t >> ld16) << (ld16 + 1)) + (t & d16m1)
                    s2 = s + d16
                    a = cand_v[pl.ds(s * L, L)]
                    b = cand_v[pl.ds(s2 * L, L)]
                    cand_v[pl.ds(s * L, L)] = jnp.minimum(a, b)
                    cand_v[pl.ds(s2 * L, L)] = jnp.maximum(a, b)
                return c
            lax.fori_loop(0, lr16, st_b, 0)

            @plsc.parallel_loop(0, nv, unroll=4)
            def _st_c(i):
                sl = pl.ds(i * L, L)
                cand_v[sl] = jnp.sort(cand_v[sl])
            return c0
        lax.fori_loop(0, n_levels, level_body, 0)

        @plsc.parallel_loop(0, N_KEEP // L, unroll=4)
        def _emit(j):
            out_v[pl.ds(j * L, L)] = lax.rev(
                cand_v[pl.ds((nv - 1 - j) * L, L)], (0,))
        pltpu.sync_copy(out_v, out_hbm.at[r])
        return 0

    lax.fori_loop(0, ROWS_PER_W, do_row, 0)


@jax.jit
def kernel(inputs):
    mesh = plsc.VectorSubcoreMesh(core_axis_name="c", subcore_axis_name="s")
    f = pl.kernel(
        _topn_body,
        out_type=jax.ShapeDtypeStruct((N_ROWS, N_KEEP), jnp.float32),
        mesh=mesh,
        compiler_params=pltpu.CompilerParams(needs_layout_passes=False),
        scratch_types=[
            pltpu.VMEM((N_COLS,), jnp.float32),
            pltpu.VMEM((N_COLS + L,), jnp.float32),
            pltpu.VMEM((NBINS,), jnp.int32),
            pltpu.VMEM((N_KEEP,), jnp.float32),
        ],
    )
    return f(inputs)


# speedup vs baseline: 22.7478x; 1.0352x over previous
import functools

import jax
import jax.numpy as jnp
from jax import lax
from jax.experimental import pallas as pl
from jax.experimental.pallas import tpu as pltpu
from jax.experimental.pallas import tpu_sc as plsc

N_ROWS = 128
N_COLS = 32768
N_KEEP = 1024
L = 16
NBINS = 2048
NV_COL = N_COLS // L
NV_BIN = NBINS // L
N_WORKERS = 32
ROWS_PER_W = N_ROWS // N_WORKERS
def _monotone_key(v):
    sign = jnp.uint32(0x80000000)
    u = lax.bitcast_convert_type(v, jnp.uint32)
    return jnp.where(u >= sign, ~u, u | sign)


def _topn_body(in_hbm, out_hbm, row_v, cand_v, hist_v, c2_v, out_v):
    wid = lax.axis_index("c") * 16 + lax.axis_index("s")
    iota = lax.iota(jnp.int32, L)
    neg_inf = jnp.float32(-jnp.inf)

    def do_row(k, _):
        r = wid * ROWS_PER_W + k
        pltpu.sync_copy(in_hbm.at[r], row_v)

        @plsc.parallel_loop(0, NV_BIN, unroll=4)
        def _clear(i):
            hist_v[pl.ds(i * L, L)] = jnp.zeros((L,), jnp.int32)

        ones = jnp.ones((L,), jnp.int32)

        @plsc.parallel_loop(0, NV_COL, unroll=8)
        def _hist(i):
            key = _monotone_key(row_v[pl.ds(i * L, L)])
            bins = (key >> jnp.uint32(21)).astype(jnp.int32)
            plsc.addupdate_scatter(hist_v, [bins], ones)

        @plsc.parallel_loop(0, NV_BIN, unroll=2,
                            carry=(jnp.int32(0), jnp.int32(-1)))
        def scan_loop(i, carry):
            acc, best = carry
            v = hist_v[pl.ds(i * L, L)]
            cs = plsc.cumsum(v)
            a_incl = N_COLS - (acc + cs - v)
            binvec = i * L + iota
            cand = jnp.where(a_incl >= N_KEEP, binvec, -1)
            return acc + jnp.max(cs), jnp.maximum(best, jnp.max(cand))
        _, bstar = scan_loop
        thresh = bstar.astype(jnp.uint32) << jnp.uint32(21)

        @plsc.parallel_loop(0, NV_COL, unroll=4, carry=jnp.int32(0))
        def comp_loop(i, off):
            v = row_v[pl.ds(i * L, L)]
            key = _monotone_key(v)
            msk = key >= thresh
            mi = msk.astype(jnp.int32)
            cs = plsc.cumsum(mi)
            idx = jnp.where(msk, off + cs - 1, N_COLS + iota)
            plsc.store_scatter(cand_v, [idx], v)
            return off + jnp.max(cs)
        m = comp_loop

        mvv = (m + L - 1) >> 4
        HB = 256
        k_rank = jnp.int32(N_KEEP)
        pref = jnp.uint32(0)
        tot = m
        for pi, shift in enumerate((24, 16, 8, 0)):
            @plsc.parallel_loop(0, (HB + L) // L, unroll=4)
            def _dclear(i):
                hist_v[pl.ds(i * L, L)] = jnp.zeros((L,), jnp.int32)

            sh = jnp.uint32(shift)

            @plsc.parallel_loop(0, mvv, unroll=2)
            def _dhist(i, pi=pi, sh=sh, pref=pref):
                key = _monotone_key(cand_v[pl.ds(i * L, L)])
                valid = (i * L + iota) < m
                if pi > 0:
                    valid = valid & ((key >> (sh + 8)) == (pref >> (sh + 8)))
                digit = ((key >> sh) & jnp.uint32(0xFF)).astype(jnp.int32)
                plsc.addupdate_scatter(hist_v, [digit], ones, mask=valid)

            @plsc.parallel_loop(0, HB // L, unroll=2,
                                carry=(jnp.int32(0), jnp.int32(-1),
                                       jnp.int32(-1)))
            def _dscan(i, carry, tot=tot, k_rank=k_rank):
                acc, bestp, bestc = carry
                v = hist_v[pl.ds(i * L, L)]
                cs = plsc.cumsum(v)
                a_incl = tot - (acc + cs - v)
                cond = a_incl >= k_rank
                dvec = (i * L + iota) << 16
                packed = jnp.where(cond, dvec + a_incl, -1)
                packc = jnp.where(cond, dvec + v, -1)
                return (acc + jnp.max(cs), jnp.maximum(bestp, jnp.max(packed)),
                        jnp.maximum(bestc, jnp.max(packc)))
            _, bestp, bestc = _dscan
            dstar = bestp >> 16
            a_star = bestp & 0xFFFF
            cntd = bestc & 0xFFFF
            k_rank = k_rank - (a_star - cntd)
            tot = cntd
            pref = pref | (dstar.astype(jnp.uint32) << sh)

        @plsc.parallel_loop(0, mvv, unroll=2, carry=jnp.int32(0))
        def comp2_loop(i, off):
            v = cand_v[pl.ds(i * L, L)]
            key = _monotone_key(v)
            msk = ((i * L + iota) < m) & (key > pref)
            mi = msk.astype(jnp.int32)
            cs = plsc.cumsum(mi)
            idx = jnp.where(msk, off + cs - 1, N_KEEP + iota)
            plsc.store_scatter(c2_v, [idx], v)
            return off + jnp.max(cs)
        c = comp2_loop

        u_t = jnp.where(pref >= jnp.uint32(0x80000000),
                        pref ^ jnp.uint32(0x80000000), ~pref)
        t_f = lax.bitcast_convert_type(jnp.broadcast_to(u_t, (L,)), jnp.float32)

        @plsc.parallel_loop(c >> 4, N_KEEP // L, unroll=2)
        def _fill(i):
            v = c2_v[pl.ds(i * L, L)]
            c2_v[pl.ds(i * L, L)] = jnp.where(i * L + iota >= c, t_f, v)

        NV = N_KEEP // L
        NPAIR = NV // 2

        @plsc.parallel_loop(0, NV, unroll=4)
        def _leaf(i):
            c2_v[pl.ds(i * L, L)] = jnp.sort(c2_v[pl.ds(i * L, L)])

        def level_body(lr16, c0):
            r16m1 = (jnp.int32(1) << lr16) - 1

            @plsc.parallel_loop(0, NPAIR, unroll=4)
            def _st_a(t):
                p = t >> lr16
                i = t & r16m1
                base = (p << (lr16 + 1)) + i
                s2 = base + ((r16m1 - i) << 1) + 1
                a = c2_v[pl.ds(base * L, L)]
                rb = lax.rev(c2_v[pl.ds(s2 * L, L)], (0,))
                c2_v[pl.ds(base * L, L)] = jnp.minimum(a, rb)
                c2_v[pl.ds(s2 * L, L)] = lax.rev(jnp.maximum(a, rb), (0,))

            def st_b(j, c):
                ld16 = lr16 - 1 - j
                d16 = jnp.int32(1) << ld16
                d16m1 = d16 - 1

                @plsc.parallel_loop(0, NPAIR, unroll=4)
                def _st_b(t):
                    s = ((t >> ld16) << (ld16 + 1)) + (t & d16m1)
                    s2 = s + d16
                    a = c2_v[pl.ds(s * L, L)]
                    b = c2_v[pl.ds(s2 * L, L)]
                    c2_v[pl.ds(s * L, L)] = jnp.minimum(a, b)
                    c2_v[pl.ds(s2 * L, L)] = jnp.maximum(a, b)
                return c
            lax.fori_loop(0, lr16, st_b, 0)

            @plsc.parallel_loop(0, NV, unroll=4)
            def _st_c(i):
                c2_v[pl.ds(i * L, L)] = jnp.sort(c2_v[pl.ds(i * L, L)])
            return c0
        lax.fori_loop(0, 6, level_body, 0)

        @plsc.parallel_loop(0, NV, unroll=4)
        def _emit(j):
            out_v[pl.ds(j * L, L)] = lax.rev(
                c2_v[pl.ds((NV - 1 - j) * L, L)], (0,))
        pltpu.sync_copy(out_v, out_hbm.at[r])
        return 0

    lax.fori_loop(0, ROWS_PER_W, do_row, 0)


@jax.jit
def kernel(inputs):
    mesh = plsc.VectorSubcoreMesh(core_axis_name="c", subcore_axis_name="s")
    f = pl.kernel(
        _topn_body,
        out_type=jax.ShapeDtypeStruct((N_ROWS, N_KEEP), jnp.float32),
        mesh=mesh,
        compiler_params=pltpu.CompilerParams(needs_layout_passes=False),
        scratch_types=[
            pltpu.VMEM((N_COLS,), jnp.float32),
            pltpu.VMEM((N_COLS + L,), jnp.float32),
            pltpu.VMEM((NBINS,), jnp.int32),
            pltpu.VMEM((N_KEEP + L,), jnp.float32),
            pltpu.VMEM((N_KEEP,), jnp.float32),
        ],
    )
    return f(inputs)


# speedup vs baseline: 23.4314x; 1.0301x over previous
import functools

import jax
import jax.numpy as jnp
from jax import lax
from jax.experimental import pallas as pl
from jax.experimental.pallas import tpu as pltpu
from jax.experimental.pallas import tpu_sc as plsc

N_ROWS = 128
N_COLS = 32768
N_KEEP = 1024
L = 16
NBINS = 2048
NV_COL = N_COLS // L
NV_BIN = NBINS // L
N_WORKERS = 32
ROWS_PER_W = N_ROWS // N_WORKERS
def _monotone_key(v):
    sign = jnp.uint32(0x80000000)
    u = lax.bitcast_convert_type(v, jnp.uint32)
    return jnp.where(u >= sign, ~u, u | sign)


def _topn_body(in_hbm, out_hbm, row_v, cand_v, hist_v, c2_v, out_v):
    wid = lax.axis_index("c") * 16 + lax.axis_index("s")
    iota = lax.iota(jnp.int32, L)
    neg_inf = jnp.float32(-jnp.inf)

    def do_row(k, _):
        r = wid * ROWS_PER_W + k
        pltpu.sync_copy(in_hbm.at[r], row_v)

        @plsc.parallel_loop(0, NV_BIN, unroll=4)
        def _clear(i):
            hist_v[pl.ds(i * L, L)] = jnp.zeros((L,), jnp.int32)

        ones = jnp.ones((L,), jnp.int32)

        @plsc.parallel_loop(0, NV_COL, unroll=16)
        def _hist(i):
            key = _monotone_key(row_v[pl.ds(i * L, L)])
            bins = (key >> jnp.uint32(21)).astype(jnp.int32)
            plsc.addupdate_scatter(hist_v, [bins], ones)

        @plsc.parallel_loop(0, NV_BIN, unroll=2,
                            carry=(jnp.int32(0), jnp.int32(-1)))
        def scan_loop(i, carry):
            acc, best = carry
            v = hist_v[pl.ds(i * L, L)]
            cs = plsc.cumsum(v)
            a_incl = N_COLS - (acc + cs - v)
            binvec = i * L + iota
            cand = jnp.where(a_incl >= N_KEEP, binvec, -1)
            return acc + jnp.max(cs), jnp.maximum(best, jnp.max(cand))
        _, bstar = scan_loop
        thresh = bstar.astype(jnp.uint32) << jnp.uint32(21)

        @plsc.parallel_loop(0, NV_COL, unroll=8, carry=jnp.int32(0))
        def comp_loop(i, off):
            v = row_v[pl.ds(i * L, L)]
            key = _monotone_key(v)
            msk = key >= thresh
            mi = msk.astype(jnp.int32)
            cs = plsc.cumsum(mi)
            idx = jnp.where(msk, off + cs - 1, N_COLS + iota)
            plsc.store_scatter(cand_v, [idx], v)
            return off + jnp.max(cs)
        m = comp_loop

        mvv = (m + L - 1) >> 4
        HB = 256
        k_rank = jnp.int32(N_KEEP)
        pref = jnp.uint32(0)
        tot = m
        for pi, shift in enumerate((24, 16, 8, 0)):
            @plsc.parallel_loop(0, (HB + L) // L, unroll=4)
            def _dclear(i):
                hist_v[pl.ds(i * L, L)] = jnp.zeros((L,), jnp.int32)

            sh = jnp.uint32(shift)

            @plsc.parallel_loop(0, mvv, unroll=2)
            def _dhist(i, pi=pi, sh=sh, pref=pref):
                key = _monotone_key(cand_v[pl.ds(i * L, L)])
                valid = (i * L + iota) < m
                if pi > 0:
                    valid = valid & ((key >> (sh + 8)) == (pref >> (sh + 8)))
                digit = ((key >> sh) & jnp.uint32(0xFF)).astype(jnp.int32)
                plsc.addupdate_scatter(hist_v, [digit], ones, mask=valid)

            @plsc.parallel_loop(0, HB // L, unroll=2,
                                carry=(jnp.int32(0), jnp.int32(-1),
                                       jnp.int32(-1)))
            def _dscan(i, carry, tot=tot, k_rank=k_rank):
                acc, bestp, bestc = carry
                v = hist_v[pl.ds(i * L, L)]
                cs = plsc.cumsum(v)
                a_incl = tot - (acc + cs - v)
                cond = a_incl >= k_rank
                dvec = (i * L + iota) << 16
                packed = jnp.where(cond, dvec + a_incl, -1)
                packc = jnp.where(cond, dvec + v, -1)
                return (acc + jnp.max(cs), jnp.maximum(bestp, jnp.max(packed)),
                        jnp.maximum(bestc, jnp.max(packc)))
            _, bestp, bestc = _dscan
            dstar = bestp >> 16
            a_star = bestp & 0xFFFF
            cntd = bestc & 0xFFFF
            k_rank = k_rank - (a_star - cntd)
            tot = cntd
            pref = pref | (dstar.astype(jnp.uint32) << sh)

        @plsc.parallel_loop(0, mvv, unroll=4, carry=jnp.int32(0))
        def comp2_loop(i, off):
            v = cand_v[pl.ds(i * L, L)]
            key = _monotone_key(v)
            msk = ((i * L + iota) < m) & (key > pref)
            mi = msk.astype(jnp.int32)
            cs = plsc.cumsum(mi)
            idx = jnp.where(msk, off + cs - 1, N_KEEP + iota)
            plsc.store_scatter(c2_v, [idx], v)
            return off + jnp.max(cs)
        c = comp2_loop

        u_t = jnp.where(pref >= jnp.uint32(0x80000000),
                        pref ^ jnp.uint32(0x80000000), ~pref)
        t_f = lax.bitcast_convert_type(jnp.broadcast_to(u_t, (L,)), jnp.float32)

        @plsc.parallel_loop(c >> 4, N_KEEP // L, unroll=2)
        def _fill(i):
            v = c2_v[pl.ds(i * L, L)]
            c2_v[pl.ds(i * L, L)] = jnp.where(i * L + iota >= c, t_f, v)

        NV = N_KEEP // L
        NPAIR = NV // 2

        @plsc.parallel_loop(0, NV, unroll=4)
        def _leaf(i):
            c2_v[pl.ds(i * L, L)] = jnp.sort(c2_v[pl.ds(i * L, L)])

        def level_body(lr16, c0):
            r16m1 = (jnp.int32(1) << lr16) - 1

            @plsc.parallel_loop(0, NPAIR, unroll=4)
            def _st_a(t):
                p = t >> lr16
                i = t & r16m1
                base = (p << (lr16 + 1)) + i
                s2 = base + ((r16m1 - i) << 1) + 1
                a = c2_v[pl.ds(base * L, L)]
                rb = lax.rev(c2_v[pl.ds(s2 * L, L)], (0,))
                c2_v[pl.ds(base * L, L)] = jnp.minimum(a, rb)
                c2_v[pl.ds(s2 * L, L)] = lax.rev(jnp.maximum(a, rb), (0,))

            def st_b(j, c):
                ld16 = lr16 - 1 - j
                d16 = jnp.int32(1) << ld16
                d16m1 = d16 - 1

                @plsc.parallel_loop(0, NPAIR, unroll=4)
                def _st_b(t):
                    s = ((t >> ld16) << (ld16 + 1)) + (t & d16m1)
                    s2 = s + d16
                    a = c2_v[pl.ds(s * L, L)]
                    b = c2_v[pl.ds(s2 * L, L)]
                    c2_v[pl.ds(s * L, L)] = jnp.minimum(a, b)
                    c2_v[pl.ds(s2 * L, L)] = jnp.maximum(a, b)
                return c
            lax.fori_loop(0, lr16, st_b, 0)

            @plsc.parallel_loop(0, NV, unroll=4)
            def _st_c(i):
                c2_v[pl.ds(i * L, L)] = jnp.sort(c2_v[pl.ds(i * L, L)])
            return c0
        lax.fori_loop(0, 6, level_body, 0)

        @plsc.parallel_loop(0, NV, unroll=4)
        def _emit(j):
            out_v[pl.ds(j * L, L)] = lax.rev(
                c2_v[pl.ds((NV - 1 - j) * L, L)], (0,))
        pltpu.sync_copy(out_v, out_hbm.at[r])
        return 0

    lax.fori_loop(0, ROWS_PER_W, do_row, 0)


@jax.jit
def kernel(inputs):
    mesh = plsc.VectorSubcoreMesh(core_axis_name="c", subcore_axis_name="s")
    f = pl.kernel(
        _topn_body,
        out_type=jax.ShapeDtypeStruct((N_ROWS, N_KEEP), jnp.float32),
        mesh=mesh,
        compiler_params=pltpu.CompilerParams(needs_layout_passes=False),
        scratch_types=[
            pltpu.VMEM((N_COLS,), jnp.float32),
            pltpu.VMEM((N_COLS + L,), jnp.float32),
            pltpu.VMEM((NBINS,), jnp.int32),
            pltpu.VMEM((N_KEEP + L,), jnp.float32),
            pltpu.VMEM((N_KEEP,), jnp.float32),
        ],
    )
    return f(inputs)
